# Initial kernel scaffold; baseline (speedup 1.0000x reference)
import jax
import jax.numpy as jnp
from jax.experimental import pallas as pl


def kernel(x, edge_index, batch, u, W_enc, b_enc, g_enc, be_enc, W1s, b1s, W2s, b2s, bn_g, bn_b, Wp, bp, gp, bpl, Wf1, bf1, gf1, bl1, Wf2, bf2, gf2, bl2, Wt, bt, Wz, bz):
    raise NotImplementedError("write your pallas kernel here")


# speedup vs baseline: 4.5039x; 4.5039x over previous
import functools

import jax
import jax.numpy as jnp
from jax import lax
from jax.experimental import pallas as pl
from jax.experimental.pallas import tpu as pltpu
from jax.experimental.pallas import tpu_sc as plsc

NC = 2
NS = 16
NW = NC * NS


@functools.lru_cache(maxsize=None)
def _make_agg_kernel(n, e, d):
    ew = e // NW
    chunk = 80
    assert e % NW == 0 and ew % chunk == 0 and n % (NS * 8) == 0
    nchunks = ew // chunk
    rows_per_tile = n // NS

    mesh = plsc.VectorSubcoreMesh(core_axis_name="c", subcore_axis_name="s")

    @functools.partial(
        pl.kernel,
        out_type=jax.ShapeDtypeStruct((NC, n, d), jnp.float32),
        mesh=mesh,
        scratch_types=[
            pltpu.VMEM((chunk,), jnp.int32),
            pltpu.VMEM((chunk,), jnp.int32),
            pltpu.VMEM((chunk, d), jnp.float32),
            pltpu.VMEM_SHARED((n, d), jnp.float32),
            pltpu.SemaphoreType.DMA,
        ],
    )
    def agg_kernel(h_hbm, src_hbm, dst_hbm, zeros_hbm, out_hbm,
                   sidx, didx, rows, acc, sem):
        c = lax.axis_index("c")
        s = lax.axis_index("s")
        pltpu.sync_copy(zeros_hbm.at[pl.ds(s * rows_per_tile, rows_per_tile)],
                        acc.at[pl.ds(s * rows_per_tile, rows_per_tile)])
        plsc.subcore_barrier()
        w = s * NC + c
        base = w * ew

        def body(i, carry):
            off = pl.multiple_of(base + i * chunk, 8)
            pltpu.sync_copy(src_hbm.at[pl.ds(off, chunk)], sidx)
            pltpu.sync_copy(dst_hbm.at[pl.ds(off, chunk)], didx)
            pltpu.async_copy(h_hbm.at[sidx], rows, sem).wait()
            pltpu.sync_copy(rows, acc.at[didx], add=True)
            return carry

        lax.fori_loop(0, nchunks, body, 0)
        plsc.subcore_barrier()
        pltpu.sync_copy(acc.at[pl.ds(s * rows_per_tile, rows_per_tile)],
                        out_hbm.at[c, pl.ds(s * rows_per_tile, rows_per_tile)])

    return agg_kernel


def _enc_body(x_ref, w_ref, b_ref, g_ref, be_ref, o_ref):
    y = jnp.dot(x_ref[...], w_ref[...], preferred_element_type=jnp.float32)
    y = y + b_ref[...]
    mu = jnp.mean(y, axis=-1, keepdims=True)
    var = jnp.mean((y - mu) * (y - mu), axis=-1, keepdims=True)
    yn = (y - mu) * lax.rsqrt(var + 1e-5) * g_ref[...] + be_ref[...]
    o_ref[...] = yn * jax.nn.sigmoid(yn)


def _mlp_body(h_ref, p0_ref, p1_ref, w1_ref, b1_ref, w2_ref, b2_ref,
              sc_ref, bi_ref, o_ref):
    m = h_ref[...] + p0_ref[0] + p1_ref[0]
    t = jnp.dot(m, w1_ref[...], preferred_element_type=jnp.float32) + b1_ref[...]
    t = t * jax.nn.sigmoid(t)
    y = jnp.dot(t, w2_ref[...], preferred_element_type=jnp.float32) + b2_ref[...]
    y = y * sc_ref[...] + bi_ref[...]
    o_ref[...] = y * jax.nn.sigmoid(y)


def _head_body(h_ref, batch_ref, u_ref, wp_ref, bp_ref, gp_ref, bpl_ref,
               wf1g_ref, wf1p_ref, bf1_ref, gf1_ref, bl1_ref,
               wf2_ref, bf2_ref, gf2_ref, bl2_ref,
               wt_ref, bt_ref, wz_ref, bz_ref,
               theta_ref, z_ref):
    nb = theta_ref.shape[0]
    n = h_ref.shape[0]
    seg = batch_ref[...]
    row_ids = lax.broadcasted_iota(jnp.int32, (nb, n), 0)
    oh = (row_ids == seg).astype(jnp.float32)
    sums = jnp.dot(oh, h_ref[...], preferred_element_type=jnp.float32)
    cnt = jnp.sum(oh, axis=-1, keepdims=True)
    h_graph = sums / jnp.maximum(cnt, 1.0)

    def ln_silu(v, g, b):
        mu = jnp.mean(v, axis=-1, keepdims=True)
        var = jnp.mean((v - mu) * (v - mu), axis=-1, keepdims=True)
        vn = (v - mu) * lax.rsqrt(var + 1e-5) * g + b
        return vn * jax.nn.sigmoid(vn)

    hp = jnp.dot(u_ref[...], wp_ref[...], preferred_element_type=jnp.float32)
    h_phys = ln_silu(hp + bp_ref[...], gp_ref[...], bpl_ref[...])

    h1 = (jnp.dot(h_graph, wf1g_ref[...], preferred_element_type=jnp.float32)
          + jnp.dot(h_phys, wf1p_ref[...], preferred_element_type=jnp.float32)
          + bf1_ref[...])
    h1 = ln_silu(h1, gf1_ref[...], bl1_ref[...])
    h2 = jnp.dot(h1, wf2_ref[...], preferred_element_type=jnp.float32) + bf2_ref[...]
    h2 = ln_silu(h2, gf2_ref[...], bl2_ref[...])
    theta_ref[...] = (jnp.sum(h2 * wt_ref[...], axis=-1, keepdims=True)
                      + bt_ref[...])
    z_ref[...] = jnp.dot(h2, wz_ref[...], preferred_element_type=jnp.float32) + bz_ref[...]


def _row2d(v):
    return v.reshape(1, -1)


def kernel(x, edge_index, batch, u, W_enc, b_enc, g_enc, be_enc, W1s, b1s,
           W2s, b2s, bn_g, bn_b, Wp, bp, gp, bpl, Wf1, bf1, gf1, bl1, Wf2,
           bf2, gf2, bl2, Wt, bt, Wz, bz):
    n, d = x.shape
    e = edge_index.shape[1]
    nb, p = u.shape
    m_dim = Wf1.shape[1]
    num_layers = W1s.shape[0]

    blk = 2000
    assert n % blk == 0
    grid = n // blk
    n_pad = ((n + NS * 8 - 1) // (NS * 8)) * (NS * 8)

    src = edge_index[0]
    dst = edge_index[1]
    zeros_nd = jnp.zeros((n_pad, d), jnp.float32)

    enc = pl.pallas_call(
        _enc_body,
        grid=(grid,),
        in_specs=[
            pl.BlockSpec((blk, d), lambda i: (i, 0)),
            pl.BlockSpec((d, d), lambda i: (0, 0)),
            pl.BlockSpec((1, d), lambda i: (0, 0)),
            pl.BlockSpec((1, d), lambda i: (0, 0)),
            pl.BlockSpec((1, d), lambda i: (0, 0)),
        ],
        out_specs=pl.BlockSpec((blk, d), lambda i: (i, 0)),
        out_shape=jax.ShapeDtypeStruct((n_pad, d), jnp.float32),
    )
    h = enc(x, W_enc, _row2d(b_enc), _row2d(g_enc), _row2d(be_enc))

    agg_fn = _make_agg_kernel(n_pad, e, d)
    bn_scale = bn_g / jnp.sqrt(1.0 + 1e-5)

    mlp = pl.pallas_call(
        _mlp_body,
        grid=(grid,),
        in_specs=[
            pl.BlockSpec((blk, d), lambda i: (i, 0)),
            pl.BlockSpec((1, blk, d), lambda i: (0, i, 0)),
            pl.BlockSpec((1, blk, d), lambda i: (1, i, 0)),
            pl.BlockSpec((d, d), lambda i: (0, 0)),
            pl.BlockSpec((1, d), lambda i: (0, 0)),
            pl.BlockSpec((d, d), lambda i: (0, 0)),
            pl.BlockSpec((1, d), lambda i: (0, 0)),
            pl.BlockSpec((1, d), lambda i: (0, 0)),
            pl.BlockSpec((1, d), lambda i: (0, 0)),
        ],
        out_specs=pl.BlockSpec((blk, d), lambda i: (i, 0)),
        out_shape=jax.ShapeDtypeStruct((n_pad, d), jnp.float32),
    )

    for i in range(num_layers):
        parts = agg_fn(h, src, dst, zeros_nd)
        h = mlp(h, parts, parts, W1s[i], _row2d(b1s[i]), W2s[i],
                _row2d(b2s[i]), _row2d(bn_scale[i]), _row2d(bn_b[i]))

    Wf1g = Wf1[:d]
    Wf1p = Wf1[d:]
    head = pl.pallas_call(
        _head_body,
        out_shape=[
            jax.ShapeDtypeStruct((nb, 1), jnp.float32),
            jax.ShapeDtypeStruct((nb, d), jnp.float32),
        ],
    )
    theta, z = head(
        h[:n], batch.reshape(1, n).astype(jnp.int32), u,
        Wp, _row2d(bp), _row2d(gp), _row2d(bpl),
        Wf1g, Wf1p, _row2d(bf1), _row2d(gf1), _row2d(bl1),
        Wf2, _row2d(bf2), _row2d(gf2), _row2d(bl2),
        _row2d(Wt.reshape(-1)), bt.reshape(1, 1), Wz, _row2d(bz))
    return (theta, z)


# speedup vs baseline: 10.5855x; 2.3503x over previous
import functools

import jax
import jax.numpy as jnp
from jax import lax
from jax.experimental import pallas as pl
from jax.experimental.pallas import tpu as pltpu
from jax.experimental.pallas import tpu_sc as plsc

NC = 2
NS = 16
NW = NC * NS


@functools.lru_cache(maxsize=None)
def _make_agg_kernel(n, e, d):
    ew = e // NW
    chunk = 80
    assert e % NW == 0 and ew % chunk == 0 and n % (NS * 8) == 0
    nchunks = ew // chunk
    rows_per_tile = n // NS

    mesh = plsc.VectorSubcoreMesh(core_axis_name="c", subcore_axis_name="s")
    npairs = nchunks // 2
    tail = nchunks - 2 * npairs

    @functools.partial(
        pl.kernel,
        out_type=jax.ShapeDtypeStruct((NC, n, d), jnp.float32),
        mesh=mesh,
        scratch_types=[
            pltpu.VMEM((nchunks, chunk), jnp.int32),
            pltpu.VMEM((2, chunk), jnp.int32),
            pltpu.VMEM((2, chunk), jnp.int32),
            pltpu.VMEM((2, chunk, d), jnp.float32),
            pltpu.VMEM_SHARED((n, d), jnp.float32),
            pltpu.SemaphoreType.DMA,
            pltpu.SemaphoreType.DMA,
        ],
    )
    def agg_kernel(h_hbm, pidx_hbm, zeros_hbm, out_hbm,
                   pidx, sidx, didx, rows, acc, sem0, sem1):
        c = lax.axis_index("c")
        s = lax.axis_index("s")
        w = s * NC + c
        pltpu.sync_copy(pidx_hbm.at[w], pidx)
        pltpu.sync_copy(zeros_hbm.at[pl.ds(s * rows_per_tile, rows_per_tile)],
                        acc.at[pl.ds(s * rows_per_tile, rows_per_tile)])
        plsc.subcore_barrier()

        sems = (sem0, sem1)

        def fire(cc, p):
            for k in range(chunk // 16):
                v = pidx[cc, pl.ds(16 * k, 16)]
                sidx[p, pl.ds(16 * k, 16)] = lax.bitwise_and(v, 0xFFFF)
                didx[p, pl.ds(16 * k, 16)] = lax.shift_right_logical(v, 16)
            pltpu.async_copy(h_hbm.at[sidx.at[p]], rows.at[p], sems[p])

        def drain(p):
            pltpu.make_async_copy(h_hbm.at[sidx.at[p]], rows.at[p],
                                  sems[p]).wait()
            pltpu.sync_copy(rows.at[p], acc.at[didx.at[p]], add=True)

        for p in range(2):
            if p < nchunks:
                fire(p, p)

        def body(j, carry):
            for p in range(2):
                cchunk = 2 * j + p
                drain(p)
                nxt = cchunk + 2

                @pl.when(nxt < nchunks)
                def _():
                    fire(nxt, p)
            return carry

        lax.fori_loop(0, npairs, body, 0)
        if tail:
            drain(0)
        plsc.subcore_barrier()
        pltpu.sync_copy(acc.at[pl.ds(s * rows_per_tile, rows_per_tile)],
                        out_hbm.at[c, pl.ds(s * rows_per_tile, rows_per_tile)])

    return agg_kernel


def _enc_body(x_ref, w_ref, b_ref, g_ref, be_ref, o_ref):
    y = jnp.dot(x_ref[...], w_ref[...], preferred_element_type=jnp.float32)
    y = y + b_ref[...]
    mu = jnp.mean(y, axis=-1, keepdims=True)
    var = jnp.mean((y - mu) * (y - mu), axis=-1, keepdims=True)
    yn = (y - mu) * lax.rsqrt(var + 1e-5) * g_ref[...] + be_ref[...]
    o_ref[...] = yn * jax.nn.sigmoid(yn)


def _mlp_body(h_ref, p0_ref, p1_ref, w1_ref, b1_ref, w2_ref, b2_ref,
              sc_ref, bi_ref, o_ref):
    m = h_ref[...] + p0_ref[0] + p1_ref[0]
    t = jnp.dot(m, w1_ref[...], preferred_element_type=jnp.float32) + b1_ref[...]
    t = t * jax.nn.sigmoid(t)
    y = jnp.dot(t, w2_ref[...], preferred_element_type=jnp.float32) + b2_ref[...]
    y = y * sc_ref[...] + bi_ref[...]
    o_ref[...] = y * jax.nn.sigmoid(y)


def _head_body(h_ref, batch_ref, u_ref, wp_ref, bp_ref, gp_ref, bpl_ref,
               wf1g_ref, wf1p_ref, bf1_ref, gf1_ref, bl1_ref,
               wf2_ref, bf2_ref, gf2_ref, bl2_ref,
               wt_ref, bt_ref, wz_ref, bz_ref,
               theta_ref, z_ref):
    nb = theta_ref.shape[0]
    n = h_ref.shape[0]
    seg = batch_ref[...]
    row_ids = lax.broadcasted_iota(jnp.int32, (nb, n), 0)
    oh = (row_ids == seg).astype(jnp.float32)
    sums = jnp.dot(oh, h_ref[...], preferred_element_type=jnp.float32)
    cnt = jnp.sum(oh, axis=-1, keepdims=True)
    h_graph = sums / jnp.maximum(cnt, 1.0)

    def ln_silu(v, g, b):
        mu = jnp.mean(v, axis=-1, keepdims=True)
        var = jnp.mean((v - mu) * (v - mu), axis=-1, keepdims=True)
        vn = (v - mu) * lax.rsqrt(var + 1e-5) * g + b
        return vn * jax.nn.sigmoid(vn)

    hp = jnp.dot(u_ref[...], wp_ref[...], preferred_element_type=jnp.float32)
    h_phys = ln_silu(hp + bp_ref[...], gp_ref[...], bpl_ref[...])

    h1 = (jnp.dot(h_graph, wf1g_ref[...], preferred_element_type=jnp.float32)
          + jnp.dot(h_phys, wf1p_ref[...], preferred_element_type=jnp.float32)
          + bf1_ref[...])
    h1 = ln_silu(h1, gf1_ref[...], bl1_ref[...])
    h2 = jnp.dot(h1, wf2_ref[...], preferred_element_type=jnp.float32) + bf2_ref[...]
    h2 = ln_silu(h2, gf2_ref[...], bl2_ref[...])
    theta_ref[...] = (jnp.sum(h2 * wt_ref[...], axis=-1, keepdims=True)
                      + bt_ref[...])
    z_ref[...] = jnp.dot(h2, wz_ref[...], preferred_element_type=jnp.float32) + bz_ref[...]


def _row2d(v):
    return v.reshape(1, -1)


def kernel(x, edge_index, batch, u, W_enc, b_enc, g_enc, be_enc, W1s, b1s,
           W2s, b2s, bn_g, bn_b, Wp, bp, gp, bpl, Wf1, bf1, gf1, bl1, Wf2,
           bf2, gf2, bl2, Wt, bt, Wz, bz):
    n, d = x.shape
    e = edge_index.shape[1]
    nb, p = u.shape
    m_dim = Wf1.shape[1]
    num_layers = W1s.shape[0]

    blk = 2000
    assert n % blk == 0
    grid = n // blk
    n_pad = ((n + NS * 8 - 1) // (NS * 8)) * (NS * 8)

    chunk = 80
    ew = e // NW
    nchunks = ew // chunk
    packed = jnp.bitwise_or(
        edge_index[0], jnp.left_shift(edge_index[1], 16)
    ).astype(jnp.int32).reshape(NW, nchunks, chunk)
    zeros_nd = jnp.zeros((n_pad, d), jnp.float32)

    enc = pl.pallas_call(
        _enc_body,
        grid=(grid,),
        in_specs=[
            pl.BlockSpec((blk, d), lambda i: (i, 0)),
            pl.BlockSpec((d, d), lambda i: (0, 0)),
            pl.BlockSpec((1, d), lambda i: (0, 0)),
            pl.BlockSpec((1, d), lambda i: (0, 0)),
            pl.BlockSpec((1, d), lambda i: (0, 0)),
        ],
        out_specs=pl.BlockSpec((blk, d), lambda i: (i, 0)),
        out_shape=jax.ShapeDtypeStruct((n_pad, d), jnp.float32),
    )
    h = enc(x, W_enc, _row2d(b_enc), _row2d(g_enc), _row2d(be_enc))

    agg_fn = _make_agg_kernel(n_pad, e, d)
    bn_scale = bn_g / jnp.sqrt(1.0 + 1e-5)

    mlp = pl.pallas_call(
        _mlp_body,
        grid=(grid,),
        in_specs=[
            pl.BlockSpec((blk, d), lambda i: (i, 0)),
            pl.BlockSpec((1, blk, d), lambda i: (0, i, 0)),
            pl.BlockSpec((1, blk, d), lambda i: (1, i, 0)),
            pl.BlockSpec((d, d), lambda i: (0, 0)),
            pl.BlockSpec((1, d), lambda i: (0, 0)),
            pl.BlockSpec((d, d), lambda i: (0, 0)),
            pl.BlockSpec((1, d), lambda i: (0, 0)),
            pl.BlockSpec((1, d), lambda i: (0, 0)),
            pl.BlockSpec((1, d), lambda i: (0, 0)),
        ],
        out_specs=pl.BlockSpec((blk, d), lambda i: (i, 0)),
        out_shape=jax.ShapeDtypeStruct((n_pad, d), jnp.float32),
    )

    for i in range(num_layers):
        parts = agg_fn(h, packed, zeros_nd)
        h = mlp(h, parts, parts, W1s[i], _row2d(b1s[i]), W2s[i],
                _row2d(b2s[i]), _row2d(bn_scale[i]), _row2d(bn_b[i]))

    Wf1g = Wf1[:d]
    Wf1p = Wf1[d:]
    head = pl.pallas_call(
        _head_body,
        out_shape=[
            jax.ShapeDtypeStruct((nb, 1), jnp.float32),
            jax.ShapeDtypeStruct((nb, d), jnp.float32),
        ],
    )
    theta, z = head(
        h[:n], batch.reshape(1, n).astype(jnp.int32), u,
        Wp, _row2d(bp), _row2d(gp), _row2d(bpl),
        Wf1g, Wf1p, _row2d(bf1), _row2d(gf1), _row2d(bl1),
        Wf2, _row2d(bf2), _row2d(gf2), _row2d(bl2),
        _row2d(Wt.reshape(-1)), bt.reshape(1, 1), Wz, _row2d(bz))
    return (theta, z)
